# baseline (speedup 1.0000x reference)
import jax
import jax.numpy as jnp
from jax import lax
from jax.experimental import pallas as pl
from jax.experimental.pallas import tpu as pltpu
from jax.experimental.pallas import tpu_sc as plsc

_B, _N, _D, _E = 4, 10000, 128, 320000
_NC, _NS = 2, 16
_NW = _NC * _NS
_C = 64
_NCHUNK = _E // _C
_CPW = _NCHUNK // _NW
_XTRA = _NCHUNK - _CPW * _NW
_SMAX = (_CPW + 1) * (_C // 2)
_R = 2


def _gather_body(table, pk_i, pk_j, bounds, out_i, out_j,
                 st_i, st_j, bnd_v, rows_i, rows_j, ib_i, ib_j, slab,
                 *sems):
    gs = (sems[0:_R], sems[_R:2 * _R])
    ss = (sems[2 * _R:3 * _R], sems[3 * _R:4 * _R])
    st = (st_i, st_j)
    rows = (rows_i, rows_j)
    ib = (ib_i, ib_j)
    outs = (out_i, out_j)

    cid = lax.axis_index("c")
    sid = lax.axis_index("s")
    wid = sid * _NC + cid
    cs = wid * _CPW + lax.min(wid, _XTRA)
    nck = jnp.where(wid < _XTRA, _CPW + 1, _CPW)
    ce = cs + nck

    pltpu.sync_copy(pk_i.at[pl.ds(cs * (_C // 2), _SMAX)], st_i)
    pltpu.sync_copy(pk_j.at[pl.ds(cs * (_C // 2), _SMAX)], st_j)
    pltpu.sync_copy(bounds, bnd_v)
    lane = lax.broadcasted_iota(jnp.int32, (16,), 0)
    bw = bnd_v[...]
    b1, b2, b3 = bw[0], bw[1], bw[2]
    bnds = (jnp.int32(0), b1, b2, b3, jnp.int32(_E))

    def unpack(ep, s, c, make_global):
        l = c - cs
        for h in range(2):
            w = st[ep][pl.ds(l * 32 + h * 16, 16)]
            lo = w & 0xFFFF
            hi = lax.shift_right_logical(w, 16)
            for q, v in ((0, lo), (1, hi)):
                if make_global:
                    eid = c * _C + h * 32 + q * 16 + lane
                    bat = (jnp.where(eid >= b1, _N, 0)
                           + jnp.where(eid >= b2, _N, 0)
                           + jnp.where(eid >= b3, _N, 0))
                    v = v + bat
                ib[ep][s, pl.ds(h * 32 + q * 16, 16)] = v

    def gather_cp(ep, s, src):
        return pltpu.make_async_copy(
            src.at[ib[ep].at[s]], rows[ep].at[s], gs[ep][s])

    def store_cp(ep, s, c):
        return pltpu.make_async_copy(
            rows[ep].at[s], outs[ep].at[pl.ds(c * _C, _C)], ss[ep][s])

    for b in range(_B):
        plsc.subcore_barrier()

        @pl.when(sid < 15)
        def _():
            pltpu.sync_copy(table.at[pl.ds(b * _N + sid * 640, 640)],
                            slab.at[pl.ds(sid * 640, 640)])

        @pl.when(sid == 15)
        def _():
            pltpu.sync_copy(table.at[pl.ds(b * _N + 9600, 400)],
                            slab.at[pl.ds(9600, 400)])

        plsc.subcore_barrier()

        c_lo = lax.max(cs, lax.shift_right_logical(bnds[b] + (_C - 1), 6))
        c_hi = lax.min(ce, lax.shift_right_logical(bnds[b + 1], 6))
        t_n = lax.max(c_hi - c_lo, 0)

        for s in range(_R):
            @pl.when(s < t_n)
            def _(s=s):
                for ep in range(2):
                    unpack(ep, s, c_lo + s, False)
                    gather_cp(ep, s, slab).start()

        def block(t, carry):
            for s in range(_R):
                k = t * _R + s
                for ep in range(2):
                    gather_cp(ep, s, slab).wait()
                    store_cp(ep, s, c_lo + k).start()
            for s in range(_R):
                k = t * _R + s
                for ep in range(2):
                    store_cp(ep, s, c_lo + k).wait()
                for ep in range(2):
                    unpack(ep, s, c_lo + k + _R, False)
                    gather_cp(ep, s, slab).start()
            return carry

        nsteady = lax.max(lax.shift_right_logical(t_n, 1) - 1, 0)
        lax.fori_loop(0, nsteady, block, 0)

        for s in range(2 * _R):
            @pl.when(nsteady * _R + s < t_n)
            def _(s=s):
                k = nsteady * _R + s
                sl = s % _R
                for ep in range(2):
                    gather_cp(ep, sl, slab).wait()
                    store_cp(ep, sl, c_lo + k).start()
                for ep in range(2):
                    store_cp(ep, sl, c_lo + k).wait()

                @pl.when(k + _R < t_n)
                def _():
                    for ep in range(2):
                        unpack(ep, sl, c_lo + k + _R, False)
                        gather_cp(ep, sl, slab).start()

    for bb in range(1, _B):
        cb = lax.shift_right_logical(bnds[bb], 6)

        @pl.when((cb >= cs) & (cb < ce))
        def _(cb=cb):
            for ep in range(2):
                unpack(ep, 0, cb, True)
                gather_cp(ep, 0, table).start()
            for ep in range(2):
                gather_cp(ep, 0, table).wait()
                store_cp(ep, 0, cb).start()
            for ep in range(2):
                store_cp(ep, 0, cb).wait()


@jax.jit
def _gather(table, pk_i, pk_j, bounds):
    mesh = plsc.VectorSubcoreMesh(
        core_axis_name="c", subcore_axis_name="s",
        num_cores=_NC, num_subcores=_NS,
    )
    return pl.kernel(
        _gather_body,
        out_type=(
            jax.ShapeDtypeStruct((_E, _D), jnp.float32),
            jax.ShapeDtypeStruct((_E, _D), jnp.float32),
        ),
        mesh=mesh,
        scratch_types=[
            pltpu.VMEM((_SMAX,), jnp.int32),
            pltpu.VMEM((_SMAX,), jnp.int32),
            pltpu.VMEM((16,), jnp.int32),
            pltpu.VMEM((_R, _C, _D), jnp.float32),
            pltpu.VMEM((_R, _C, _D), jnp.float32),
            pltpu.VMEM((_R, _C), jnp.int32),
            pltpu.VMEM((_R, _C), jnp.int32),
            pltpu.VMEM_SHARED((_N, _D), jnp.float32),
        ] + [pltpu.SemaphoreType.DMA] * (4 * _R),
    )(table, pk_i, pk_j, bounds)


def _pack(col):
    a = col.reshape(_E // 32, 2, 16)
    pk = (a[:, 0, :] | (a[:, 1, :] << 16)).reshape(_E // 2)
    return jnp.concatenate([pk, jnp.zeros(32, jnp.int32)])


def kernel(inputs, selected_edges):
    table = inputs.reshape(_B * _N, _D)
    pk_i = _pack(selected_edges[:, 1])
    pk_j = _pack(selected_edges[:, 2])
    bounds = jnp.zeros(16, jnp.int32).at[:3].set(
        jnp.searchsorted(selected_edges[:, 0], jnp.arange(1, 4)
                         ).astype(jnp.int32))
    return _gather(table, pk_i, pk_j, bounds)


# speedup vs baseline: 1.2222x; 1.2222x over previous
import jax
import jax.numpy as jnp
from jax import lax
from jax.experimental import pallas as pl
from jax.experimental.pallas import tpu as pltpu
from jax.experimental.pallas import tpu_sc as plsc

_B, _N, _D, _E = 4, 10000, 128, 320000
_NC, _NS = 2, 16
_NW = _NC * _NS
_C = 32
_NCHUNK = _E // _C
_CPW = _NCHUNK // _NW
_XTRA = _NCHUNK - _CPW * _NW
_SMAX = (_CPW + 1) * _C
_R = 3


def _gather_body(table, vi_f, vj_f, bounds, out_i, out_j,
                 st_i, st_j, bnd_v, rows_i, rows_j, ib_i, ib_j, slab,
                 *sems):
    gs = (sems[0:_R], sems[_R:2 * _R])
    ss = (sems[2 * _R:3 * _R], sems[3 * _R:4 * _R])
    st = (st_i, st_j)
    rows = (rows_i, rows_j)
    ib = (ib_i, ib_j)
    outs = (out_i, out_j)

    cid = lax.axis_index("c")
    sid = lax.axis_index("s")
    wid = sid * _NC + cid
    cs = wid * _CPW + lax.min(wid, _XTRA)
    ce = cs + _CPW + jnp.where(wid < _XTRA, 1, 0)

    pltpu.sync_copy(vi_f.at[pl.ds(cs * _C, _SMAX)], st_i)
    pltpu.sync_copy(vj_f.at[pl.ds(cs * _C, _SMAX)], st_j)
    pltpu.sync_copy(bounds, bnd_v)
    lane = lax.broadcasted_iota(jnp.int32, (16,), 0)
    bw = bnd_v[...]
    b1, b2, b3 = bw[0], bw[1], bw[2]
    bnds = (jnp.int32(0), b1, b2, b3, jnp.int32(_E))

    def gather_cp(ep, s, c, src):
        return pltpu.make_async_copy(
            src.at[st[ep].at[pl.ds((c - cs) * _C, _C)]], rows[ep].at[s],
            gs[ep][s])

    def gather_glb_cp(ep, s):
        return pltpu.make_async_copy(
            table.at[ib[ep]], rows[ep].at[s], gs[ep][s])

    def store_cp(ep, s, c):
        return pltpu.make_async_copy(
            rows[ep].at[s], outs[ep].at[pl.ds(c * _C, _C)], ss[ep][s])

    for b in range(_B):
        plsc.subcore_barrier()

        @pl.when(sid < 15)
        def _():
            pltpu.sync_copy(table.at[pl.ds(b * _N + sid * 640, 640)],
                            slab.at[pl.ds(sid * 640, 640)])

        @pl.when(sid == 15)
        def _():
            pltpu.sync_copy(table.at[pl.ds(b * _N + 9600, 400)],
                            slab.at[pl.ds(9600, 400)])

        plsc.subcore_barrier()

        c_lo = lax.max(cs, lax.shift_right_logical(bnds[b] + (_C - 1), 5))
        c_hi = lax.min(ce, lax.shift_right_logical(bnds[b + 1], 5))
        t_n = lax.max(c_hi - c_lo, 0)

        for s in range(_R):
            @pl.when(s < t_n)
            def _(s=s):
                for ep in range(2):
                    gather_cp(ep, s, c_lo + s, slab).start()

        def block(t, carry):
            for s in range(_R):
                k = t * _R + s
                for ep in range(2):
                    gather_cp(ep, s, c_lo + k, slab).wait()
                    store_cp(ep, s, c_lo + k).start()
            for s in range(_R):
                k = t * _R + s
                for ep in range(2):
                    store_cp(ep, s, c_lo + k).wait()
                for ep in range(2):
                    gather_cp(ep, s, c_lo + k + _R, slab).start()
            return carry

        nsteady = lax.max(lax.div(t_n - _R, _R), 0)
        lax.fori_loop(0, nsteady, block, 0)

        for s in range(2 * _R):
            @pl.when(nsteady * _R + s < t_n)
            def _(s=s):
                k = nsteady * _R + s
                sl = s % _R
                for ep in range(2):
                    gather_cp(ep, sl, c_lo + k, slab).wait()
                    store_cp(ep, sl, c_lo + k).start()
                for ep in range(2):
                    store_cp(ep, sl, c_lo + k).wait()

                @pl.when(k + _R < t_n)
                def _():
                    for ep in range(2):
                        gather_cp(ep, sl, c_lo + k + _R, slab).start()

    for bb in range(1, _B):
        cb = lax.shift_right_logical(bnds[bb], 5)

        @pl.when((cb >= cs) & (cb < ce))
        def _(cb=cb):
            l = cb - cs
            for ep in range(2):
                for q in range(2):
                    v = st[ep][pl.ds(l * _C + q * 16, 16)]
                    eid = cb * _C + q * 16 + lane
                    bat = (jnp.where(eid >= b1, _N, 0)
                           + jnp.where(eid >= b2, _N, 0)
                           + jnp.where(eid >= b3, _N, 0))
                    ib[ep][pl.ds(q * 16, 16)] = v + bat
                gather_glb_cp(ep, 0).start()
            for ep in range(2):
                gather_glb_cp(ep, 0).wait()
                store_cp(ep, 0, cb).start()
            for ep in range(2):
                store_cp(ep, 0, cb).wait()


@jax.jit
def _gather(table, vi_f, vj_f, bounds):
    mesh = plsc.VectorSubcoreMesh(
        core_axis_name="c", subcore_axis_name="s",
        num_cores=_NC, num_subcores=_NS,
    )
    return pl.kernel(
        _gather_body,
        out_type=(
            jax.ShapeDtypeStruct((_E, _D), jnp.float32),
            jax.ShapeDtypeStruct((_E, _D), jnp.float32),
        ),
        mesh=mesh,
        scratch_types=[
            pltpu.VMEM((_SMAX,), jnp.int32),
            pltpu.VMEM((_SMAX,), jnp.int32),
            pltpu.VMEM((16,), jnp.int32),
            pltpu.VMEM((_R, _C, _D), jnp.float32),
            pltpu.VMEM((_R, _C, _D), jnp.float32),
            pltpu.VMEM((_C,), jnp.int32),
            pltpu.VMEM((_C,), jnp.int32),
            pltpu.VMEM_SHARED((_N, _D), jnp.float32),
        ] + [pltpu.SemaphoreType.DMA] * (4 * _R),
    )(table, vi_f, vj_f, bounds)


def kernel(inputs, selected_edges):
    table = inputs.reshape(_B * _N, _D)
    pad = jnp.zeros(_C, jnp.int32)
    vi_f = jnp.concatenate([selected_edges[:, 1], pad])
    vj_f = jnp.concatenate([selected_edges[:, 2], pad])
    bounds = jnp.zeros(16, jnp.int32).at[:3].set(
        jnp.searchsorted(selected_edges[:, 0], jnp.arange(1, 4)
                         ).astype(jnp.int32))
    return _gather(table, vi_f, vj_f, bounds)


# speedup vs baseline: 2.2045x; 1.8037x over previous
import jax
import jax.numpy as jnp
from jax import lax
from jax.experimental import pallas as pl
from jax.experimental.pallas import tpu as pltpu
from jax.experimental.pallas import tpu_sc as plsc

_B, _N, _D, _E = 4, 10000, 128, 320000
_NC, _NS = 2, 16
_NW = _NC * _NS
_EPW = _E // _NW
_C = 80
_R = 5
_NFULL = _EPW // _C
_NBLK = _NFULL // _R
_TAIL = _EPW - _C


def _gather_body(table, idx_i, idx_j, out_i, out_j,
                 idx_i_v, idx_j_v, rows_i, rows_j, *sems):
    gs = (sems[0:_R], sems[_R:2 * _R])
    ss = (sems[2 * _R:3 * _R], sems[3 * _R:4 * _R])
    idx_v = (idx_i_v, idx_j_v)
    rows = (rows_i, rows_j)
    outs = (out_i, out_j)

    wid = lax.axis_index("s") * _NC + lax.axis_index("c")
    ebase = wid * _EPW
    pltpu.sync_copy(idx_i.at[wid], idx_i_v)
    pltpu.sync_copy(idx_j.at[wid], idx_j_v)

    def gather_cp(ep, b, off):
        return pltpu.make_async_copy(
            table.at[idx_v[ep].at[pl.ds(off, _C)]], rows[ep].at[b],
            gs[ep][b])

    def store_cp(ep, b, off):
        return pltpu.make_async_copy(
            rows[ep].at[b], outs[ep].at[pl.ds(ebase + off, _C)], ss[ep][b])

    for b in range(_R):
        for ep in range(2):
            gather_cp(ep, b, b * _C).start()

    def block(t, carry):
        cps = []
        for b in range(_R):
            off = (t * _R + b) * _C
            for ep in range(2):
                gather_cp(ep, b, off).wait()
                cp = store_cp(ep, b, off)
                cp.start()
                cps.append(cp)
        for b in range(_R):
            for ep in range(2):
                cps[2 * b + ep].wait()

            @pl.when(t < _NBLK - 1)
            def _():
                off2 = ((t + 1) * _R + b) * _C
                for ep in range(2):
                    gather_cp(ep, b, off2).start()
        return carry

    lax.fori_loop(0, _NBLK, block, 0)

    for ep in range(2):
        gather_cp(ep, 0, _TAIL).start()
    for ep in range(2):
        gather_cp(ep, 0, _TAIL).wait()
        store_cp(ep, 0, _TAIL).start()
    for ep in range(2):
        store_cp(ep, 0, _TAIL).wait()


@jax.jit
def _gather(table, idx_i, idx_j):
    mesh = plsc.VectorSubcoreMesh(
        core_axis_name="c", subcore_axis_name="s",
        num_cores=_NC, num_subcores=_NS,
    )
    return pl.kernel(
        _gather_body,
        out_type=(
            jax.ShapeDtypeStruct((_E, _D), jnp.float32),
            jax.ShapeDtypeStruct((_E, _D), jnp.float32),
        ),
        mesh=mesh,
        scratch_types=[
            pltpu.VMEM((_EPW,), jnp.int32),
            pltpu.VMEM((_EPW,), jnp.int32),
            pltpu.VMEM((_R, _C, _D), jnp.float32),
            pltpu.VMEM((_R, _C, _D), jnp.float32),
        ] + [pltpu.SemaphoreType.DMA] * (4 * _R),
    )(table, idx_i, idx_j)


def kernel(inputs, selected_edges):
    table = inputs.reshape(_B * _N, _D)
    idx_i = selected_edges[:, 4].reshape(_NW, _EPW)
    idx_j = selected_edges[:, 5].reshape(_NW, _EPW)
    return _gather(table, idx_i, idx_j)


# speedup vs baseline: 2.2228x; 1.0083x over previous
import jax
import jax.numpy as jnp
from jax import lax
from jax.experimental import pallas as pl
from jax.experimental.pallas import tpu as pltpu
from jax.experimental.pallas import tpu_sc as plsc

_B, _N, _D, _E = 4, 10000, 128, 320000
_NC, _NS = 2, 16
_NW = _NC * _NS
_EPW = _E // _NW
_C = 80
_R = 5
_NFULL = _EPW // _C
_NBLK = _NFULL // _R


def _gather_body(table, idx_i, idx_j, out_i, out_j,
                 idx_i_v, idx_j_v, rows_i, rows_j, *sems):
    gs = (sems[0:_R], sems[_R:2 * _R])
    ss = (sems[2 * _R:3 * _R], sems[3 * _R:4 * _R])
    idx_v = (idx_i_v, idx_j_v)
    rows = (rows_i, rows_j)
    outs = (out_i, out_j)

    wid = lax.axis_index("s") * _NC + lax.axis_index("c")
    ebase = wid * _EPW
    pltpu.sync_copy(idx_i.at[wid], idx_i_v)
    pltpu.sync_copy(idx_j.at[wid], idx_j_v)

    def gather_cp(ep, b, off):
        return pltpu.make_async_copy(
            table.at[idx_v[ep].at[pl.ds(off, _C)]], rows[ep].at[b],
            gs[ep][b])

    def store_cp(ep, b, off):
        return pltpu.make_async_copy(
            rows[ep].at[b], outs[ep].at[pl.ds(ebase + off, _C)], ss[ep][b])

    for b in range(_R):
        for ep in range(2):
            gather_cp(ep, b, b * _C).start()

    def block(t, carry):
        cps = []
        for b in range(_R):
            off = (t * _R + b) * _C
            for ep in range(2):
                gather_cp(ep, b, off).wait()
                cp = store_cp(ep, b, off)
                cp.start()
                cps.append(cp)
        for b in range(_R):
            for ep in range(2):
                cps[2 * b + ep].wait()

            @pl.when(t < _NBLK - 1)
            def _():
                off2 = ((t + 1) * _R + b) * _C
                for ep in range(2):
                    gather_cp(ep, b, off2).start()
        return carry

    lax.fori_loop(0, _NBLK, block, 0)


@jax.jit
def _gather(table, idx_i, idx_j):
    mesh = plsc.VectorSubcoreMesh(
        core_axis_name="c", subcore_axis_name="s",
        num_cores=_NC, num_subcores=_NS,
    )
    return pl.kernel(
        _gather_body,
        out_type=(
            jax.ShapeDtypeStruct((_E, _D), jnp.float32),
            jax.ShapeDtypeStruct((_E, _D), jnp.float32),
        ),
        mesh=mesh,
        scratch_types=[
            pltpu.VMEM((_EPW,), jnp.int32),
            pltpu.VMEM((_EPW,), jnp.int32),
            pltpu.VMEM((_R, _C, _D), jnp.float32),
            pltpu.VMEM((_R, _C, _D), jnp.float32),
        ] + [pltpu.SemaphoreType.DMA] * (4 * _R),
    )(table, idx_i, idx_j)


def kernel(inputs, selected_edges):
    table = inputs.reshape(_B * _N, _D)
    idx_i = selected_edges[:, 4].reshape(_NW, _EPW)
    idx_j = selected_edges[:, 5].reshape(_NW, _EPW)
    return _gather(table, idx_i, idx_j)
